# baseline (speedup 1.0000x reference)
import jax
import jax.numpy as jnp
from jax.experimental import pallas as pl

_IB = 1024


def _agg_cast_body(ht_ref, hn_ref, adj_ref, zt_ref, adjbf_ref):
    hb = hn_ref[...].astype(jnp.bfloat16)
    ab = adj_ref[...].astype(jnp.bfloat16)
    adjbf_ref[...] = ab
    zt_ref[...] = ht_ref[...] + jax.lax.dot_general(
        hb, ab, (((0,), (0,)), ((), ())),
        preferred_element_type=jnp.float32)


def _agg_bf_body(ht_ref, hn_ref, adjbf_ref, zt_ref):
    hb = hn_ref[...].astype(jnp.bfloat16)
    zt_ref[...] = ht_ref[...] + jax.lax.dot_general(
        hb, adjbf_ref[...], (((0,), (0,)), ((), ())),
        preferred_element_type=jnp.float32)


def _mlp_body(zt_ref, w1t_ref, b1_ref, g_ref, be_ref, w2t_ref, b2_ref,
              ht_ref, hn_ref, p_ref):
    z = zt_ref[...]
    u = jnp.dot(w1t_ref[...], z,
                preferred_element_type=jnp.float32) + b1_ref[...]
    mu = jnp.mean(u, axis=1, keepdims=True)
    d = u - mu
    var = jnp.mean(d * d, axis=1, keepdims=True)
    y = g_ref[...] * d * jax.lax.rsqrt(var + 1e-5) + be_ref[...]
    y = jnp.maximum(y, 0.0)
    h = jnp.dot(w2t_ref[...], y,
                preferred_element_type=jnp.float32) + b2_ref[...]
    h = jnp.maximum(h, 0.0)
    ht_ref[...] = h
    hn_ref[...] = h.T
    p_ref[...] = jnp.mean(h, axis=1, keepdims=True)


def _head_body(p1_ref, p2_ref, p3_ref, w1t_ref, b1_ref, w2t_ref, b2_ref,
               out_ref):
    p = jnp.concatenate([p1_ref[...], p2_ref[...], p3_ref[...]], axis=0)
    t = jnp.dot(w1t_ref[...], p,
                preferred_element_type=jnp.float32) + b1_ref[...]
    t = jnp.maximum(t, 0.0)
    o = jnp.dot(w2t_ref[...], t,
                preferred_element_type=jnp.float32) + b2_ref[...]
    m = jnp.max(o, axis=1, keepdims=True)
    out_ref[...] = o - m - jnp.log(
        jnp.sum(jnp.exp(o - m), axis=1, keepdims=True))


def kernel(features, adj, c1_W1, c1_b1, c1_g, c1_be, c1_W2, c1_b2,
           c2_W1, c2_b1, c2_g, c2_be, c2_W2, c2_b2,
           c3_W1, c3_b1, c3_g, c3_be, c3_W2, c3_b2,
           m_W1, m_b1, m_W2, m_b2):
    n, dim = features.shape
    h = c1_W1.shape[1]
    ib = _IB

    ib1 = 256
    agg_cast = pl.pallas_call(
        _agg_cast_body,
        grid=(pl.cdiv(n, ib1),),
        in_specs=[
            pl.BlockSpec((dim, ib1), lambda i: (0, i)),
            pl.BlockSpec((n, dim), lambda i: (0, 0)),
            pl.BlockSpec((n, ib1), lambda i: (0, i)),
        ],
        out_specs=(
            pl.BlockSpec((dim, ib1), lambda i: (0, i)),
            pl.BlockSpec((n, ib1), lambda i: (0, i)),
        ),
        out_shape=(
            jax.ShapeDtypeStruct((dim, n), jnp.float32),
            jax.ShapeDtypeStruct((n, n), jnp.bfloat16),
        ),
    )

    agg_bf = pl.pallas_call(
        _agg_bf_body,
        grid=(pl.cdiv(n, ib),),
        in_specs=[
            pl.BlockSpec((dim, ib), lambda i: (0, i)),
            pl.BlockSpec((n, dim), lambda i: (0, 0)),
            pl.BlockSpec((n, ib), lambda i: (0, i)),
        ],
        out_specs=pl.BlockSpec((dim, ib), lambda i: (0, i)),
        out_shape=jax.ShapeDtypeStruct((dim, n), jnp.float32),
    )

    mlp = pl.pallas_call(
        _mlp_body,
        out_shape=(
            jax.ShapeDtypeStruct((h, n), jnp.float32),
            jax.ShapeDtypeStruct((n, h), jnp.float32),
            jax.ShapeDtypeStruct((h, 1), jnp.float32),
        ),
    )

    head = pl.pallas_call(
        _head_body,
        out_shape=jax.ShapeDtypeStruct((1, 1), jnp.float32),
    )

    def mlp_call(zt, W1, b1, g, be, W2, b2):
        return mlp(zt, W1.T, b1[:, None], g[:, None], be[:, None],
                   W2.T, b2[:, None])

    ht0 = features.T
    z1t, adj_bf = agg_cast(ht0, features, adj)
    h1t, h1n, p1 = mlp_call(z1t, c1_W1, c1_b1, c1_g, c1_be, c1_W2, c1_b2)
    z2t = agg_bf(h1t, h1n, adj_bf)
    h2t, h2n, p2 = mlp_call(z2t, c2_W1, c2_b1, c2_g, c2_be, c2_W2, c2_b2)
    z3t = agg_bf(h2t, h2n, adj_bf)
    _, _, p3 = mlp_call(z3t, c3_W1, c3_b1, c3_g, c3_be, c3_W2, c3_b2)
    out = head(p1, p2, p3, m_W1.T, m_b1[:, None], m_W2.T, m_b2[:, None])
    return out[0]
